# baseline (speedup 1.0000x reference)
import functools

import jax
import jax.numpy as jnp
from jax import lax
from jax.experimental import pallas as pl
from jax.experimental.pallas import tpu as pltpu
from jax.experimental.pallas import tpu_sc as plsc

NUM_SEQ = 16384
T = 50
D = 64
H = 8
B = 4096
S = H * H
ROW = T * D

NC = 2
NS = 16
NW = NC * NS
CHUNK = 32

NB = 256


def _make_sc_gather(nrows):
    mesh = plsc.VectorSubcoreMesh(core_axis_name="c", subcore_axis_name="s")
    b_per_w = nrows // NW

    @functools.partial(
        pl.kernel,
        mesh=mesh,
        out_type=[
            jax.ShapeDtypeStruct((nrows, ROW), jnp.float32),
            jax.ShapeDtypeStruct((nrows,), jnp.int32),
        ],
        scratch_types=[
            pltpu.VMEM((b_per_w,), jnp.int32),
            pltpu.VMEM((CHUNK, ROW), jnp.float32),
            pltpu.VMEM((b_per_w,), jnp.int32),
            pltpu.SemaphoreType.DMA,
            pltpu.SemaphoreType.DMA,
        ],
    )
    def gather_k(table_hbm, idx_hbm, lens_hbm, y_out, lens_out,
                 idx_v, rows_v, lens_loc, sem_r, sem_l):
        wid = lax.axis_index("s") * NC + lax.axis_index("c")
        base = wid * b_per_w
        pltpu.sync_copy(idx_hbm.at[pl.ds(base, b_per_w)], idx_v)
        cp_l = pltpu.async_copy(lens_hbm.at[idx_v], lens_loc, sem_l)
        cp_l.wait()
        pltpu.sync_copy(lens_loc, lens_out.at[pl.ds(base, b_per_w)])
        for c in range(b_per_w // CHUNK):
            off = base + c * CHUNK
            cp_r = pltpu.async_copy(
                table_hbm.at[idx_v.at[pl.ds(c * CHUNK, CHUNK)]], rows_v, sem_r)
            cp_r.wait()
            pltpu.sync_copy(rows_v, y_out.at[pl.ds(off, CHUNK)])

    return gather_k


def _fwd_body(y_ref, len_ref, K_ref, init_ref, ET2_ref, bias2_ref, out_ref):
    ET2 = ET2_ref[...]
    Km = K_ref[...]
    bias2 = bias2_ref[...]
    lens = len_ref[...]

    def emit(p):
        yp = y_ref[:, 2 * p * D:(2 * p + 2) * D].astype(jnp.bfloat16)
        lb = jnp.dot(yp, ET2, preferred_element_type=jnp.float32) + bias2
        c = jnp.max(lb, axis=-1, keepdims=True)
        return c, jnp.exp(lb - c)

    def trans(alpha):
        return jnp.dot(alpha.astype(jnp.bfloat16), Km,
                       preferred_element_type=jnp.float32)

    c, eb = emit(0)
    alpha = init_ref[...] * eb[:, 0:S]
    ll = c
    act = lens > 1
    alpha = jnp.where(act, trans(alpha) * eb[:, S:2 * S], alpha)
    ll = jnp.where(act, ll + c, ll)
    for p in range(1, T // 2):
        c, eb = emit(p)
        act = lens > 2 * p
        alpha = jnp.where(act, trans(alpha) * eb[:, 0:S], alpha)
        ll = jnp.where(act, ll + c, ll)
        act = lens > 2 * p + 1
        alpha = jnp.where(act, trans(alpha) * eb[:, S:2 * S], alpha)
        ll = jnp.where(act, ll + c, ll)
        if p % 2 == 1:
            s = jnp.sum(alpha, axis=-1, keepdims=True)
            alpha = alpha * (1.0 / s)
            ll = ll + jnp.log(s)
    s = jnp.sum(alpha, axis=-1, keepdims=True)
    out_ref[...] = ll + jnp.log(s)


def _make_tc_compute(nrows, interpret=False):
    return pl.pallas_call(
        _fwd_body,
        grid=(nrows // NB,),
        in_specs=[
            pl.BlockSpec((NB, ROW), lambda i: (i, 0)),
            pl.BlockSpec((NB, 1), lambda i: (i, 0)),
            pl.BlockSpec((S, S), lambda i: (0, 0)),
            pl.BlockSpec((1, S), lambda i: (0, 0)),
            pl.BlockSpec((2 * D, 2 * S), lambda i: (0, 0)),
            pl.BlockSpec((1, 2 * S), lambda i: (0, 0)),
        ],
        out_specs=pl.BlockSpec((NB, 1), lambda i: (i, 0)),
        out_shape=jax.ShapeDtypeStruct((nrows, 1), jnp.float32),
        interpret=interpret,
    )


def kernel(sequences, lengths, mb, mask, probs_w, w_init, probs_x, x_init,
           probs_y):
    eps = 1e-6
    pw = probs_w + eps
    pw = pw / pw.sum(-1, keepdims=True)
    px = probs_x + eps
    px = px / px.sum(-1, keepdims=True)
    pwi = w_init + eps
    pwi = pwi / pwi.sum()
    pxi = x_init + eps
    pxi = pxi / pxi.sum()
    py = jnp.clip(probs_y, eps, 1.0 - eps)
    lpy = jnp.log(py)
    l1m = jnp.log1p(-py)
    ET = (lpy - l1m).reshape(S, D).T
    ET2 = jnp.zeros((2 * D, 2 * S), jnp.float32)
    ET2 = ET2.at[:D, :S].set(ET).at[D:, S:].set(ET).astype(jnp.bfloat16)
    bias = l1m.sum(-1).reshape(1, S)
    bias2 = jnp.concatenate([bias, bias], axis=1)
    Km = (pw[:, None, :, None] * px[None, :, None, :]).reshape(S, S)
    Km = Km.astype(jnp.bfloat16)
    init = (pwi[:, None] * pxi[None, :]).reshape(1, S)

    table = sequences.reshape(NUM_SEQ, ROW)
    y_g, lens_g = _make_sc_gather(B)(table, mb.astype(jnp.int32),
                                     lengths.astype(jnp.int32))
    ll = _make_tc_compute(B)(y_g, lens_g[:, None], Km, init, ET2, bias2)
    return jnp.where(mask, ll[:, 0], 0.0)


# speedup vs baseline: 1.1335x; 1.1335x over previous
import functools

import jax
import jax.numpy as jnp
from jax import lax
from jax.experimental import pallas as pl
from jax.experimental.pallas import tpu as pltpu
from jax.experimental.pallas import tpu_sc as plsc

NUM_SEQ = 16384
T = 50
D = 64
H = 8
B = 4096
S = H * H
ROW = T * D

NC = 2
NS = 16
NW = NC * NS
CHUNK = 32

NB = 1024


def _make_sc_gather(nrows):
    mesh = plsc.VectorSubcoreMesh(core_axis_name="c", subcore_axis_name="s")
    b_per_w = nrows // NW

    @functools.partial(
        pl.kernel,
        mesh=mesh,
        out_type=[
            jax.ShapeDtypeStruct((nrows, ROW), jnp.float32),
            jax.ShapeDtypeStruct((nrows,), jnp.int32),
        ],
        scratch_types=[
            pltpu.VMEM((b_per_w,), jnp.int32),
            pltpu.VMEM((CHUNK, ROW), jnp.float32),
            pltpu.VMEM((b_per_w,), jnp.int32),
            pltpu.SemaphoreType.DMA,
            pltpu.SemaphoreType.DMA,
        ],
    )
    def gather_k(table_hbm, idx_hbm, lens_hbm, y_out, lens_out,
                 idx_v, rows_v, lens_loc, sem_r, sem_l):
        wid = lax.axis_index("s") * NC + lax.axis_index("c")
        base = wid * b_per_w
        pltpu.sync_copy(idx_hbm.at[pl.ds(base, b_per_w)], idx_v)
        cp_l = pltpu.async_copy(lens_hbm.at[idx_v], lens_loc, sem_l)
        cp_l.wait()
        pltpu.sync_copy(lens_loc, lens_out.at[pl.ds(base, b_per_w)])
        for c in range(b_per_w // CHUNK):
            off = base + c * CHUNK
            cp_r = pltpu.async_copy(
                table_hbm.at[idx_v.at[pl.ds(c * CHUNK, CHUNK)]], rows_v, sem_r)
            cp_r.wait()
            pltpu.sync_copy(rows_v, y_out.at[pl.ds(off, CHUNK)])

    return gather_k


def _fwd_body(y_ref, len_ref, K_ref, init_ref, ET2_ref, bias2_ref, out_ref):
    ET2 = ET2_ref[...]
    Km = K_ref[...]
    bias2 = bias2_ref[...]
    lens = len_ref[...]

    def emit(p):
        yp = y_ref[:, 2 * p * D:(2 * p + 2) * D].astype(jnp.bfloat16)
        lb = jnp.dot(yp, ET2, preferred_element_type=jnp.float32) + bias2
        c = jnp.max(lb, axis=-1, keepdims=True)
        return c, jnp.exp(lb - c)

    def trans(alpha):
        return jnp.dot(alpha.astype(jnp.bfloat16), Km,
                       preferred_element_type=jnp.float32)

    c, eb = emit(0)
    alpha = init_ref[...] * eb[:, 0:S]
    ll = c
    act = lens > 1
    alpha = jnp.where(act, trans(alpha) * eb[:, S:2 * S], alpha)
    ll = jnp.where(act, ll + c, ll)
    for p in range(1, T // 2):
        c, eb = emit(p)
        act = lens > 2 * p
        alpha = jnp.where(act, trans(alpha) * eb[:, 0:S], alpha)
        ll = jnp.where(act, ll + c, ll)
        act = lens > 2 * p + 1
        alpha = jnp.where(act, trans(alpha) * eb[:, S:2 * S], alpha)
        ll = jnp.where(act, ll + c, ll)
        if p % 2 == 1:
            s = jnp.sum(alpha, axis=-1, keepdims=True)
            alpha = alpha * (1.0 / s)
            ll = ll + jnp.log(s)
    s = jnp.sum(alpha, axis=-1, keepdims=True)
    out_ref[...] = ll + jnp.log(s)


def _make_tc_compute(nrows, interpret=False):
    return pl.pallas_call(
        _fwd_body,
        grid=(nrows // NB,),
        in_specs=[
            pl.BlockSpec((NB, ROW), lambda i: (i, 0)),
            pl.BlockSpec((NB, 1), lambda i: (i, 0)),
            pl.BlockSpec((S, S), lambda i: (0, 0)),
            pl.BlockSpec((1, S), lambda i: (0, 0)),
            pl.BlockSpec((2 * D, 2 * S), lambda i: (0, 0)),
            pl.BlockSpec((1, 2 * S), lambda i: (0, 0)),
        ],
        out_specs=pl.BlockSpec((NB, 1), lambda i: (i, 0)),
        out_shape=jax.ShapeDtypeStruct((nrows, 1), jnp.float32),
        interpret=interpret,
    )


def kernel(sequences, lengths, mb, mask, probs_w, w_init, probs_x, x_init,
           probs_y):
    eps = 1e-6
    pw = probs_w + eps
    pw = pw / pw.sum(-1, keepdims=True)
    px = probs_x + eps
    px = px / px.sum(-1, keepdims=True)
    pwi = w_init + eps
    pwi = pwi / pwi.sum()
    pxi = x_init + eps
    pxi = pxi / pxi.sum()
    py = jnp.clip(probs_y, eps, 1.0 - eps)
    lpy = jnp.log(py)
    l1m = jnp.log1p(-py)
    ET = (lpy - l1m).reshape(S, D).T
    ET2 = jnp.zeros((2 * D, 2 * S), jnp.float32)
    ET2 = ET2.at[:D, :S].set(ET).at[D:, S:].set(ET).astype(jnp.bfloat16)
    bias = l1m.sum(-1).reshape(1, S)
    bias2 = jnp.concatenate([bias, bias], axis=1)
    Km = (pw[:, None, :, None] * px[None, :, None, :]).reshape(S, S)
    Km = Km.astype(jnp.bfloat16)
    init = (pwi[:, None] * pxi[None, :]).reshape(1, S)

    table = sequences.reshape(NUM_SEQ, ROW)
    y_g, lens_g = _make_sc_gather(B)(table, mb.astype(jnp.int32),
                                     lengths.astype(jnp.int32))
    ll = _make_tc_compute(B)(y_g, lens_g[:, None], Km, init, ET2, bias2)
    return jnp.where(mask, ll[:, 0], 0.0)
